# baseline (speedup 1.0000x reference)
import functools

import jax
import jax.numpy as jnp
from jax import lax
from jax.experimental import pallas as pl
from jax.experimental.pallas import tpu as pltpu
from jax.experimental.pallas import tpu_sc as plsc

NO_ELECTRON_WEIGHT = 0.1
SALIENCE_ALPHA = 0.25
SALIENCE_GAMMA = 2.0

B = 4
P = 4096
N = 512
CHUNK = 2048
NC = P // CHUNK
ROWS = CHUNK // 8

MASK_ELEMS = float(B * P * N)
SAL_ELEMS = float(B * 65536)
DICE_SLOTS = float(B * N)
NQ = 2048.0

LOG1P_COEF = (
    9.47330713874095e-10, 0.9999997699016518, -0.4999906247526394,
    0.33318192091874266, -0.24872052845702441, 0.1935175008521293,
    -0.1453396423814142, 0.0947555638867925, -0.04705113527250597,
    0.015055349789856167, -0.0022609953752676533,
)

NW = 32
SAL_TOTAL = B * 65536
SAL_PER_W = SAL_TOTAL // NW
SAL_VECS = SAL_PER_W // 16


def _bce(logits, targets):
    return (jnp.maximum(logits, 0.0) - logits * targets
            + jnp.log1p(jnp.exp(-jnp.abs(logits))))


def _mask_body(pred_ref, lab_ref, mask_ref, true_ref, pos_ref, post_ref,
               out_ref, acc_ref, bce_ref, spt_ref, sp_ref, st_ref):
    b = pl.program_id(0)
    c = pl.program_id(1)
    first = jnp.logical_and(b == 0, c == 0)

    @pl.when(first)
    def _init():
        acc_ref[0] = 0.0
        bce_ref[...] = jnp.zeros((8, N), jnp.float32)

    zero = jnp.zeros((8, N), jnp.float32)
    abce, apt, ap, at = zero, zero, zero, zero
    for i in range(ROWS):
        x = mask_ref[0, i * 8:(i + 1) * 8, :]
        t = true_ref[0, i * 8:(i + 1) * 8, :]
        e = jnp.exp(-jnp.abs(x))
        u = 1.0 + e
        l1p = jnp.log(u)
        r = 1.0 / u
        probs = jnp.where(x >= 0.0, r, e * r)
        abce = abce + (jnp.maximum(x, 0.0) - x * t + l1p)
        apt = apt + probs * t
        ap = ap + probs
        at = at + t

    bce_ref[...] += abce

    @pl.when(c == 0)
    def _dice_init():
        spt_ref[...] = apt
        sp_ref[...] = ap
        st_ref[...] = at

    @pl.when(c > 0)
    def _dice_acc():
        spt_ref[...] += apt
        sp_ref[...] += ap
        st_ref[...] += at

    @pl.when(c == NC - 1)
    def _dice_done():
        num = jnp.sum(spt_ref[...], axis=0, keepdims=True)
        den = (jnp.sum(sp_ref[...], axis=0, keepdims=True)
               + jnp.sum(st_ref[...], axis=0, keepdims=True))
        dice = 1.0 - (2.0 * num + 1.0) / (den + 1.0)
        acc_ref[0] += jnp.sum(dice) * (1.0 / DICE_SLOTS)

    @pl.when(first)
    def _small():
        lab = lab_ref[...].astype(jnp.float32)
        w = jnp.where(lab == 1.0, 1.0, NO_ELECTRON_WEIGHT)
        per_q = _bce(pred_ref[...], lab)
        acc_ref[0] += jnp.sum(w * per_q) / jnp.sum(w)

        d = pos_ref[...] - post_ref[...]
        a = jnp.abs(d)
        h = jnp.where(a < 1.0, 0.5 * d * d, a - 0.5)
        acc_ref[0] += jnp.sum(h) * (1.0 / NQ)

    @pl.when(jnp.logical_and(b == B - 1, c == NC - 1))
    def _emit():
        total = acc_ref[0] + jnp.sum(bce_ref[...]) * (1.0 / MASK_ELEMS)
        out_ref[...] = jnp.broadcast_to(total, (1, 1))


def _mask_losses(pred2, lab2, mask_logits, true_masks, posp, post):
    return pl.pallas_call(
        _mask_body,
        grid=(B, NC),
        in_specs=[
            pl.BlockSpec((16, 128), lambda b, c: (0, 0)),
            pl.BlockSpec((16, 128), lambda b, c: (0, 0)),
            pl.BlockSpec((1, CHUNK, N), lambda b, c: (b, c, 0)),
            pl.BlockSpec((1, CHUNK, N), lambda b, c: (b, c, 0)),
            pl.BlockSpec((32, 128), lambda b, c: (0, 0)),
            pl.BlockSpec((32, 128), lambda b, c: (0, 0)),
        ],
        out_specs=pl.BlockSpec((1, 1), lambda b, c: (0, 0)),
        out_shape=jax.ShapeDtypeStruct((1, 1), jnp.float32),
        scratch_shapes=[
            pltpu.SMEM((1,), jnp.float32),
            pltpu.VMEM((8, N), jnp.float32),
            pltpu.VMEM((8, N), jnp.float32),
            pltpu.VMEM((8, N), jnp.float32),
            pltpu.VMEM((8, N), jnp.float32),
        ],
        compiler_params=pltpu.CompilerParams(
            dimension_semantics=("arbitrary", "arbitrary"),
        ),
    )(pred2, lab2, mask_logits, true_masks, posp, post)


def _sal_body(s_hbm, t_hbm, out_hbm, s_v, t_v, acc_v):
    wid = lax.axis_index("s") * 2 + lax.axis_index("c")
    base = wid * SAL_PER_W
    pltpu.sync_copy(s_hbm.at[pl.ds(base, SAL_PER_W)], s_v)
    pltpu.sync_copy(t_hbm.at[pl.ds(base, SAL_PER_W)], t_v)

    def body(i, acc):
        s = s_v[pl.ds(i * 16, 16)]
        tt = t_v[pl.ds(i * 16, 16)]
        e = jnp.exp(-jnp.abs(s))
        u = 1.0 + e
        r = 1.0 / u
        p = jnp.where(s >= 0.0, r, e * r)
        l1p = jnp.full((16,), LOG1P_COEF[-1], jnp.float32)
        for cf in LOG1P_COEF[-2::-1]:
            l1p = l1p * e + cf
        ce = jnp.maximum(s, 0.0) - s * tt + l1p
        p_t = p * tt + (1.0 - p) * (1.0 - tt)
        om = 1.0 - p_t
        alpha_t = SALIENCE_ALPHA * tt + (1.0 - SALIENCE_ALPHA) * (1.0 - tt)
        return acc + alpha_t * ce * om * om

    acc = lax.fori_loop(0, SAL_VECS, body, jnp.zeros((16,), jnp.float32))
    acc_v[...] = acc
    pltpu.sync_copy(acc_v, out_hbm.at[wid])


def _salience_partials(sal_flat, salt_flat):
    mesh = plsc.VectorSubcoreMesh(core_axis_name="c", subcore_axis_name="s")
    fn = functools.partial(
        pl.kernel, mesh=mesh,
        out_type=jax.ShapeDtypeStruct((NW, 16), jnp.float32),
        scratch_types=[
            pltpu.VMEM((SAL_PER_W,), jnp.float32),
            pltpu.VMEM((SAL_PER_W,), jnp.float32),
            pltpu.VMEM((16,), jnp.float32),
        ],
    )(_sal_body)
    return fn(sal_flat, salt_flat)


def _combine_body(a_ref, b_ref, out_ref):
    out_ref[...] = a_ref[...] + jnp.sum(b_ref[...]) * (1.0 / SAL_ELEMS)


def _combine(a, bparts):
    return pl.pallas_call(
        _combine_body,
        out_shape=jax.ShapeDtypeStruct((1, 1), jnp.float32),
    )(a, bparts)


@jax.jit
def kernel(pred_logits, labels, mask_logits, true_masks, pred_positions,
           true_positions, salience_logits, salience_targets):
    pred2 = pred_logits.reshape(16, 128)
    lab2 = labels.reshape(16, 128)
    posp = pred_positions.reshape(32, 128)
    post = true_positions.reshape(32, 128)

    sal_part = _salience_partials(salience_logits.reshape(-1),
                                  salience_targets.reshape(-1))
    a = _mask_losses(pred2, lab2, mask_logits, true_masks, posp, post)
    return _combine(a, sal_part).reshape(())


# speedup vs baseline: 1.0009x; 1.0009x over previous
import functools

import jax
import jax.numpy as jnp
from jax import lax
from jax.experimental import pallas as pl
from jax.experimental.pallas import tpu as pltpu
from jax.experimental.pallas import tpu_sc as plsc

NO_ELECTRON_WEIGHT = 0.1
SALIENCE_ALPHA = 0.25
SALIENCE_GAMMA = 2.0

B = 4
P = 4096
N = 512
CHUNK = 2048
NC = P // CHUNK
ROWS = CHUNK // 8

MASK_ELEMS = float(B * P * N)
SAL_ELEMS = float(B * 65536)
DICE_SLOTS = float(B * N)
NQ = 2048.0

LOG1P_COEF = (
    9.47330713874095e-10, 0.9999997699016518, -0.4999906247526394,
    0.33318192091874266, -0.24872052845702441, 0.1935175008521293,
    -0.1453396423814142, 0.0947555638867925, -0.04705113527250597,
    0.015055349789856167, -0.0022609953752676533,
)

NW = 32
SAL_TOTAL = B * 65536
SAL_PER_W = SAL_TOTAL // NW
SAL_VECS = SAL_PER_W // 16


def _bce(logits, targets):
    return (jnp.maximum(logits, 0.0) - logits * targets
            + jnp.log1p(jnp.exp(-jnp.abs(logits))))


def _mask_body(pred_ref, lab_ref, mask_ref, true_ref, pos_ref, post_ref,
               out_ref, acc_ref, bce_ref, spt_ref, sp_ref, st_ref):
    b = pl.program_id(0)
    c = pl.program_id(1)
    first = jnp.logical_and(b == 0, c == 0)

    @pl.when(first)
    def _init():
        acc_ref[0] = 0.0
        bce_ref[...] = jnp.zeros((8, N), jnp.float32)

    zero = jnp.zeros((8, N), jnp.float32)
    abce, apt, ap, at = zero, zero, zero, zero
    for i in range(ROWS):
        x = mask_ref[0, i * 8:(i + 1) * 8, :]
        t = true_ref[0, i * 8:(i + 1) * 8, :]
        e = jnp.exp(-jnp.abs(x))
        u = 1.0 + e
        l1p = jnp.log(u)
        r = 1.0 / u
        probs = jnp.where(x >= 0.0, r, e * r)
        abce = abce + (jnp.maximum(x, 0.0) - x * t + l1p)
        apt = apt + probs * t
        ap = ap + probs
        at = at + t

    bce_ref[...] += abce

    @pl.when(c == 0)
    def _dice_init():
        spt_ref[...] = apt
        sp_ref[...] = ap
        st_ref[...] = at

    @pl.when(c > 0)
    def _dice_acc():
        spt_ref[...] += apt
        sp_ref[...] += ap
        st_ref[...] += at

    @pl.when(c == NC - 1)
    def _dice_done():
        num = jnp.sum(spt_ref[...], axis=0, keepdims=True)
        den = (jnp.sum(sp_ref[...], axis=0, keepdims=True)
               + jnp.sum(st_ref[...], axis=0, keepdims=True))
        dice = 1.0 - (2.0 * num + 1.0) / (den + 1.0)
        acc_ref[0] += jnp.sum(dice) * (1.0 / DICE_SLOTS)

    @pl.when(first)
    def _small():
        lab = lab_ref[...].astype(jnp.float32)
        w = jnp.where(lab == 1.0, 1.0, NO_ELECTRON_WEIGHT)
        per_q = _bce(pred_ref[...], lab)
        acc_ref[0] += jnp.sum(w * per_q) / jnp.sum(w)

        d = pos_ref[...] - post_ref[...]
        a = jnp.abs(d)
        h = jnp.where(a < 1.0, 0.5 * d * d, a - 0.5)
        acc_ref[0] += jnp.sum(h) * (1.0 / NQ)

    @pl.when(jnp.logical_and(b == B - 1, c == NC - 1))
    def _emit():
        total = acc_ref[0] + jnp.sum(bce_ref[...]) * (1.0 / MASK_ELEMS)
        out_ref[...] = jnp.broadcast_to(total, (1, 1))


def _mask_losses(pred2, lab2, mask_logits, true_masks, posp, post):
    return pl.pallas_call(
        _mask_body,
        grid=(B, NC),
        in_specs=[
            pl.BlockSpec((16, 128), lambda b, c: (0, 0)),
            pl.BlockSpec((16, 128), lambda b, c: (0, 0)),
            pl.BlockSpec((1, CHUNK, N), lambda b, c: (b, c, 0)),
            pl.BlockSpec((1, CHUNK, N), lambda b, c: (b, c, 0)),
            pl.BlockSpec((32, 128), lambda b, c: (0, 0)),
            pl.BlockSpec((32, 128), lambda b, c: (0, 0)),
        ],
        out_specs=pl.BlockSpec((1, 1), lambda b, c: (0, 0)),
        out_shape=jax.ShapeDtypeStruct((1, 1), jnp.float32),
        scratch_shapes=[
            pltpu.SMEM((1,), jnp.float32),
            pltpu.VMEM((8, N), jnp.float32),
            pltpu.VMEM((8, N), jnp.float32),
            pltpu.VMEM((8, N), jnp.float32),
            pltpu.VMEM((8, N), jnp.float32),
        ],
        compiler_params=pltpu.CompilerParams(
            dimension_semantics=("arbitrary", "arbitrary"),
        ),
    )(pred2, lab2, mask_logits, true_masks, posp, post)


def _sal_body(s_hbm, t_hbm, out_hbm, s_v, t_v, acc_v):
    wid = lax.axis_index("s") * 2 + lax.axis_index("c")
    base = wid * SAL_PER_W
    pltpu.sync_copy(s_hbm.at[pl.ds(base, SAL_PER_W)], s_v)
    pltpu.sync_copy(t_hbm.at[pl.ds(base, SAL_PER_W)], t_v)

    UNROLL = 8

    def one(s, tt):
        e = jnp.exp(-jnp.abs(s))
        u = 1.0 + e
        r = 1.0 / u
        p = jnp.where(s >= 0.0, r, e * r)
        l1p = jnp.full((16,), LOG1P_COEF[-1], jnp.float32)
        for cf in LOG1P_COEF[-2::-1]:
            l1p = l1p * e + cf
        ce = jnp.maximum(s, 0.0) - s * tt + l1p
        p_t = p * tt + (1.0 - p) * (1.0 - tt)
        om = 1.0 - p_t
        alpha_t = SALIENCE_ALPHA * tt + (1.0 - SALIENCE_ALPHA) * (1.0 - tt)
        return alpha_t * ce * om * om

    def body(i, accs):
        out = []
        for j in range(UNROLL):
            off = (i * UNROLL + j) * 16
            contrib = one(s_v[pl.ds(off, 16)], t_v[pl.ds(off, 16)])
            out.append(accs[j] + contrib)
        return tuple(out)

    zeros = tuple(jnp.zeros((16,), jnp.float32) for _ in range(UNROLL))
    accs = lax.fori_loop(0, SAL_VECS // UNROLL, body, zeros)
    acc = accs[0]
    for j in range(1, UNROLL):
        acc = acc + accs[j]
    acc_v[...] = acc
    pltpu.sync_copy(acc_v, out_hbm.at[wid])


def _salience_partials(sal_flat, salt_flat):
    mesh = plsc.VectorSubcoreMesh(core_axis_name="c", subcore_axis_name="s")
    fn = functools.partial(
        pl.kernel, mesh=mesh,
        out_type=jax.ShapeDtypeStruct((NW, 16), jnp.float32),
        scratch_types=[
            pltpu.VMEM((SAL_PER_W,), jnp.float32),
            pltpu.VMEM((SAL_PER_W,), jnp.float32),
            pltpu.VMEM((16,), jnp.float32),
        ],
    )(_sal_body)
    return fn(sal_flat, salt_flat)


def _combine_body(a_ref, b_ref, out_ref):
    out_ref[...] = a_ref[...] + jnp.sum(b_ref[...]) * (1.0 / SAL_ELEMS)


def _combine(a, bparts):
    return pl.pallas_call(
        _combine_body,
        out_shape=jax.ShapeDtypeStruct((1, 1), jnp.float32),
    )(a, bparts)


@jax.jit
def kernel(pred_logits, labels, mask_logits, true_masks, pred_positions,
           true_positions, salience_logits, salience_targets):
    pred2 = pred_logits.reshape(16, 128)
    lab2 = labels.reshape(16, 128)
    posp = pred_positions.reshape(32, 128)
    post = true_positions.reshape(32, 128)

    sal_part = _salience_partials(salience_logits.reshape(-1),
                                  salience_targets.reshape(-1))
    a = _mask_losses(pred2, lab2, mask_logits, true_masks, posp, post)
    return _combine(a, sal_part).reshape(())


# speedup vs baseline: 1.3836x; 1.3823x over previous
import jax
import jax.numpy as jnp
from jax.experimental import pallas as pl
from jax.experimental.pallas import tpu as pltpu

NO_ELECTRON_WEIGHT = 0.1
SALIENCE_ALPHA = 0.25
SALIENCE_GAMMA = 2.0

B = 4
P = 4096
N = 512
CHUNK = 2048
NC = P // CHUNK
ROWS = CHUNK // 8

MASK_ELEMS = float(B * P * N)
SAL_ELEMS = float(B * 65536)
DICE_SLOTS = float(B * N)
NQ = 2048.0


def _bce(logits, targets):
    return (jnp.maximum(logits, 0.0) - logits * targets
            + jnp.log1p(jnp.exp(-jnp.abs(logits))))


def _loss_body(pred_ref, lab_ref, mask_ref, true_ref, pos_ref, post_ref,
               sal_ref, salt_ref, out_ref, acc_ref, bce_ref, spt_ref,
               sp_ref, st_ref):
    b = pl.program_id(0)
    c = pl.program_id(1)
    first = jnp.logical_and(b == 0, c == 0)

    @pl.when(first)
    def _init():
        acc_ref[0] = 0.0
        bce_ref[...] = jnp.zeros((8, N), jnp.float32)

    zero = jnp.zeros((8, N), jnp.float32)
    abce, apt, ap, at = zero, zero, zero, zero
    for i in range(ROWS):
        x = mask_ref[0, i * 8:(i + 1) * 8, :]
        t = true_ref[0, i * 8:(i + 1) * 8, :]
        e = jnp.exp(-jnp.abs(x))
        u = 1.0 + e
        l1p = jnp.log(u)
        r = 1.0 / u
        probs = jnp.where(x >= 0.0, r, e * r)
        abce = abce + (jnp.maximum(x, 0.0) - x * t + l1p)
        apt = apt + probs * t
        ap = ap + probs
        at = at + t

    bce_ref[...] += abce

    @pl.when(c == 0)
    def _dice_init():
        spt_ref[...] = apt
        sp_ref[...] = ap
        st_ref[...] = at

    @pl.when(c > 0)
    def _dice_acc():
        spt_ref[...] += apt
        sp_ref[...] += ap
        st_ref[...] += at

    @pl.when(c == NC - 1)
    def _dice_done():
        num = jnp.sum(spt_ref[...], axis=0, keepdims=True)
        den = (jnp.sum(sp_ref[...], axis=0, keepdims=True)
               + jnp.sum(st_ref[...], axis=0, keepdims=True))
        dice = 1.0 - (2.0 * num + 1.0) / (den + 1.0)
        acc_ref[0] += jnp.sum(dice) * (1.0 / DICE_SLOTS)

    @pl.when(c == 0)
    def _salience():
        sacc = jnp.zeros((32, 128), jnp.float32)
        for i in range(16):
            s = sal_ref[0, i * 32:(i + 1) * 32, :]
            tt = salt_ref[0, i * 32:(i + 1) * 32, :]
            es = jnp.exp(-jnp.abs(s))
            us = 1.0 + es
            rs = 1.0 / us
            p = jnp.where(s >= 0.0, rs, es * rs)
            ce = jnp.maximum(s, 0.0) - s * tt + jnp.log(us)
            p_t = p * tt + (1.0 - p) * (1.0 - tt)
            om = 1.0 - p_t
            alpha_t = SALIENCE_ALPHA * tt + (1.0 - SALIENCE_ALPHA) * (1.0 - tt)
            sacc = sacc + alpha_t * ce * om * om
        acc_ref[0] += jnp.sum(sacc) * (1.0 / SAL_ELEMS)

    @pl.when(first)
    def _small():
        lab = lab_ref[...].astype(jnp.float32)
        w = jnp.where(lab == 1.0, 1.0, NO_ELECTRON_WEIGHT)
        per_q = _bce(pred_ref[...], lab)
        acc_ref[0] += jnp.sum(w * per_q) / jnp.sum(w)

        d = pos_ref[...] - post_ref[...]
        a = jnp.abs(d)
        h = jnp.where(a < 1.0, 0.5 * d * d, a - 0.5)
        acc_ref[0] += jnp.sum(h) * (1.0 / NQ)

    @pl.when(jnp.logical_and(b == B - 1, c == NC - 1))
    def _emit():
        total = acc_ref[0] + jnp.sum(bce_ref[...]) * (1.0 / MASK_ELEMS)
        out_ref[...] = jnp.broadcast_to(total, (1, 1))


@jax.jit
def kernel(pred_logits, labels, mask_logits, true_masks, pred_positions,
           true_positions, salience_logits, salience_targets):
    pred2 = pred_logits.reshape(16, 128)
    lab2 = labels.reshape(16, 128)
    posp = pred_positions.reshape(32, 128)
    post = true_positions.reshape(32, 128)
    sal3 = salience_logits.reshape(B, 512, 128)
    salt3 = salience_targets.reshape(B, 512, 128)

    grid = (B, NC)
    out = pl.pallas_call(
        _loss_body,
        grid=grid,
        in_specs=[
            pl.BlockSpec((16, 128), lambda b, c: (0, 0)),
            pl.BlockSpec((16, 128), lambda b, c: (0, 0)),
            pl.BlockSpec((1, CHUNK, N), lambda b, c: (b, c, 0)),
            pl.BlockSpec((1, CHUNK, N), lambda b, c: (b, c, 0)),
            pl.BlockSpec((32, 128), lambda b, c: (0, 0)),
            pl.BlockSpec((32, 128), lambda b, c: (0, 0)),
            pl.BlockSpec((1, 512, 128), lambda b, c: (b, 0, 0)),
            pl.BlockSpec((1, 512, 128), lambda b, c: (b, 0, 0)),
        ],
        out_specs=pl.BlockSpec((1, 1), lambda b, c: (0, 0)),
        out_shape=jax.ShapeDtypeStruct((1, 1), jnp.float32),
        scratch_shapes=[
            pltpu.SMEM((1,), jnp.float32),
            pltpu.VMEM((8, N), jnp.float32),
            pltpu.VMEM((8, N), jnp.float32),
            pltpu.VMEM((8, N), jnp.float32),
            pltpu.VMEM((8, N), jnp.float32),
        ],
        compiler_params=pltpu.CompilerParams(
            dimension_semantics=("arbitrary", "arbitrary"),
        ),
    )(pred2, lab2, mask_logits, true_masks, posp, post, sal3, salt3)
    return out.reshape(())


# speedup vs baseline: 1.4261x; 1.0308x over previous
import jax
import jax.numpy as jnp
from jax.experimental import pallas as pl
from jax.experimental.pallas import tpu as pltpu

NO_ELECTRON_WEIGHT = 0.1
SALIENCE_ALPHA = 0.25
SALIENCE_GAMMA = 2.0

B = 4
P = 4096
N = 512
CHUNK = 2048
NC = P // CHUNK
ROWS = CHUNK // 8

MASK_ELEMS = float(B * P * N)
SAL_ELEMS = float(B * 65536)
DICE_SLOTS = float(B * N)
NQ = 2048.0


def _bce(logits, targets):
    return (jnp.maximum(logits, 0.0) - logits * targets
            + jnp.log1p(jnp.exp(-jnp.abs(logits))))


def _loss_body(pred_ref, lab_ref, mask_ref, true_ref, pos_ref, post_ref,
               sal_ref, salt_ref, out_ref, acc_ref, bce_ref, spt_ref,
               sp_ref, st_ref):
    b = pl.program_id(0)
    c = pl.program_id(1)
    first = jnp.logical_and(b == 0, c == 0)

    @pl.when(first)
    def _init():
        acc_ref[0] = 0.0
        bce_ref[...] = jnp.zeros((8, N), jnp.float32)

    zero = jnp.zeros((8, N), jnp.float32)
    abce, apt, ap, at = zero, zero, zero, zero
    for i in range(ROWS):
        x = mask_ref[0, i * 8:(i + 1) * 8, :]
        t = true_ref[0, i * 8:(i + 1) * 8, :]
        e = jnp.exp(-x)
        p = 1.0 / (1.0 + e)
        pm = jnp.maximum(p, 1.0 - p)
        abce = abce + (jnp.maximum(x, 0.0) - x * t - jnp.log(pm))
        apt = apt + p * t
        ap = ap + p
        at = at + t

    bce_ref[...] += abce

    @pl.when(c == 0)
    def _dice_init():
        spt_ref[...] = apt
        sp_ref[...] = ap
        st_ref[...] = at

    @pl.when(c > 0)
    def _dice_acc():
        spt_ref[...] += apt
        sp_ref[...] += ap
        st_ref[...] += at

    @pl.when(c == NC - 1)
    def _dice_done():
        num = jnp.sum(spt_ref[...], axis=0, keepdims=True)
        den = (jnp.sum(sp_ref[...], axis=0, keepdims=True)
               + jnp.sum(st_ref[...], axis=0, keepdims=True))
        dice = 1.0 - (2.0 * num + 1.0) / (den + 1.0)
        acc_ref[0] += jnp.sum(dice) * (1.0 / DICE_SLOTS)

    @pl.when(c == 0)
    def _salience():
        sacc = jnp.zeros((32, 128), jnp.float32)
        for i in range(16):
            s = sal_ref[0, i * 32:(i + 1) * 32, :]
            tt = salt_ref[0, i * 32:(i + 1) * 32, :]
            es = jnp.exp(-s)
            p = 1.0 / (1.0 + es)
            pms = jnp.maximum(p, 1.0 - p)
            ce = jnp.maximum(s, 0.0) - s * tt - jnp.log(pms)
            p_t = p * tt + (1.0 - p) * (1.0 - tt)
            om = 1.0 - p_t
            alpha_t = SALIENCE_ALPHA * tt + (1.0 - SALIENCE_ALPHA) * (1.0 - tt)
            sacc = sacc + alpha_t * ce * om * om
        acc_ref[0] += jnp.sum(sacc) * (1.0 / SAL_ELEMS)

    @pl.when(first)
    def _small():
        lab = lab_ref[...].astype(jnp.float32)
        w = jnp.where(lab == 1.0, 1.0, NO_ELECTRON_WEIGHT)
        per_q = _bce(pred_ref[...], lab)
        acc_ref[0] += jnp.sum(w * per_q) / jnp.sum(w)

        d = pos_ref[...] - post_ref[...]
        a = jnp.abs(d)
        h = jnp.where(a < 1.0, 0.5 * d * d, a - 0.5)
        acc_ref[0] += jnp.sum(h) * (1.0 / NQ)

    @pl.when(jnp.logical_and(b == B - 1, c == NC - 1))
    def _emit():
        total = acc_ref[0] + jnp.sum(bce_ref[...]) * (1.0 / MASK_ELEMS)
        out_ref[...] = jnp.broadcast_to(total, (1, 1))


@jax.jit
def kernel(pred_logits, labels, mask_logits, true_masks, pred_positions,
           true_positions, salience_logits, salience_targets):
    pred2 = pred_logits.reshape(16, 128)
    lab2 = labels.reshape(16, 128)
    posp = pred_positions.reshape(32, 128)
    post = true_positions.reshape(32, 128)
    sal3 = salience_logits.reshape(B, 512, 128)
    salt3 = salience_targets.reshape(B, 512, 128)

    grid = (B, NC)
    out = pl.pallas_call(
        _loss_body,
        grid=grid,
        in_specs=[
            pl.BlockSpec((16, 128), lambda b, c: (0, 0)),
            pl.BlockSpec((16, 128), lambda b, c: (0, 0)),
            pl.BlockSpec((1, CHUNK, N), lambda b, c: (b, c, 0)),
            pl.BlockSpec((1, CHUNK, N), lambda b, c: (b, c, 0)),
            pl.BlockSpec((32, 128), lambda b, c: (0, 0)),
            pl.BlockSpec((32, 128), lambda b, c: (0, 0)),
            pl.BlockSpec((1, 512, 128), lambda b, c: (b, 0, 0)),
            pl.BlockSpec((1, 512, 128), lambda b, c: (b, 0, 0)),
        ],
        out_specs=pl.BlockSpec((1, 1), lambda b, c: (0, 0)),
        out_shape=jax.ShapeDtypeStruct((1, 1), jnp.float32),
        scratch_shapes=[
            pltpu.SMEM((1,), jnp.float32),
            pltpu.VMEM((8, N), jnp.float32),
            pltpu.VMEM((8, N), jnp.float32),
            pltpu.VMEM((8, N), jnp.float32),
            pltpu.VMEM((8, N), jnp.float32),
        ],
        compiler_params=pltpu.CompilerParams(
            dimension_semantics=("arbitrary", "arbitrary"),
        ),
    )(pred2, lab2, mask_logits, true_masks, posp, post, sal3, salt3)
    return out.reshape(())
